# baseline (speedup 1.0000x reference)
import jax
import jax.numpy as jnp
from jax.experimental import pallas as pl
from jax.experimental.pallas import tpu as pltpu

_B = 16384
_A = 51
_BLOCK = 2048


def _ce_body(x_ref, t_ref, out_ref):
    x = x_ref[...]
    t = t_ref[...]
    m = jnp.max(x, axis=1, keepdims=True)
    s = jnp.sum(jnp.exp(x - m), axis=1, keepdims=True)
    lse = jnp.log(s) + m
    partial = jnp.sum(t * (lse - x)) * (1.0 / _B)

    @pl.when(pl.program_id(0) == 0)
    def _():
        out_ref[0, 0] = 0.0

    out_ref[0, 0] += partial


def kernel(current_logits, target_distribution):
    n = _B // _BLOCK
    out = pl.pallas_call(
        _ce_body,
        grid=(n,),
        in_specs=[
            pl.BlockSpec((_BLOCK, _A), lambda i: (i, 0)),
            pl.BlockSpec((_BLOCK, _A), lambda i: (i, 0)),
        ],
        out_specs=pl.BlockSpec(memory_space=pltpu.SMEM),
        out_shape=jax.ShapeDtypeStruct((1, 1), jnp.float32),
    )(current_logits, target_distribution)
    return out[0, 0]
